# Initial kernel scaffold; baseline (speedup 1.0000x reference)
import jax
import jax.numpy as jnp
from jax.experimental import pallas as pl


def kernel(mu, antecedents):
    raise NotImplementedError("write your pallas kernel here")


# speedup vs baseline: 53.1172x; 53.1172x over previous
import jax
import jax.numpy as jnp
from jax import lax
from jax.experimental import pallas as pl
from jax.experimental.pallas import tpu as pltpu

_B, _V, _M, _R = 1024, 16, 8, 4096
_RBLK = 512
_TINY = 1e-30


def _fire_block(muT_ref, aT_ref, out_ref):
    lmu = jnp.log(jnp.maximum(muT_ref[...], _TINY))
    aT = aT_ref[...]
    a_tiled = jnp.concatenate([aT] * _M, axis=0)
    m_of_row = lax.broadcasted_iota(jnp.int32, (_M * _V, _RBLK), 0) // _V
    oh = (a_tiled == m_of_row).astype(jnp.float32)
    acc = lax.dot_general(
        lmu, oh, (((1,), (0,)), ((), ())),
        precision=lax.Precision.HIGHEST,
        preferred_element_type=jnp.float32,
    )
    out_ref[...] = jnp.exp(acc)


def kernel(mu, antecedents):
    batch_shape = mu.shape[:-2]
    mu = jnp.reshape(mu, (-1, _V, _M))
    b = mu.shape[0]
    muT = jnp.swapaxes(mu, 1, 2).reshape(b, _M * _V)
    aT = antecedents.T
    out = pl.pallas_call(
        _fire_block,
        grid=(_R // _RBLK,),
        in_specs=[
            pl.BlockSpec((b, _M * _V), lambda j: (0, 0)),
            pl.BlockSpec((_V, _RBLK), lambda j: (0, j)),
        ],
        out_specs=pl.BlockSpec((b, _RBLK), lambda j: (0, j)),
        out_shape=jax.ShapeDtypeStruct((b, _R), jnp.float32),
    )(muT, aT)
    return jnp.reshape(out, (*batch_shape, _R))
